# baseline (speedup 1.0000x reference)
import jax
import jax.numpy as jnp
from jax.experimental import pallas as pl

_RR = 256


def _add_kernel(x_ref, pe_ref, o_ref):
    o_ref[...] = x_ref[...] + pe_ref[...]


def kernel(x, pos_emb):
    B, T, D = x.shape
    N = T * D
    xt = x.transpose(1, 2, 0).reshape(N, B)
    pe = pos_emb.reshape(N, 1)
    out = pl.pallas_call(
        _add_kernel,
        grid=(N // _RR,),
        in_specs=[
            pl.BlockSpec((_RR, B), lambda i: (i, 0)),
            pl.BlockSpec((_RR, 1), lambda i: (i, 0)),
        ],
        out_specs=pl.BlockSpec((_RR, B), lambda i: (i, 0)),
        out_shape=jax.ShapeDtypeStruct((N, B), x.dtype),
    )(xt, pe)
    return out.reshape(T, D, B).transpose(2, 0, 1)


# speedup vs baseline: 1.0113x; 1.0113x over previous
import jax
import jax.numpy as jnp
from jax.experimental import pallas as pl

_RR = 640


def _add_kernel(x_ref, pe_ref, o_ref):
    o_ref[...] = x_ref[...] + pe_ref[...]


def kernel(x, pos_emb):
    B, T, D = x.shape
    N = T * D
    xt = x.transpose(1, 2, 0).reshape(N, B)
    pe = pos_emb.reshape(N, 1)
    out = pl.pallas_call(
        _add_kernel,
        grid=(N // _RR,),
        in_specs=[
            pl.BlockSpec((_RR, B), lambda i: (i, 0)),
            pl.BlockSpec((_RR, 1), lambda i: (i, 0)),
        ],
        out_specs=pl.BlockSpec((_RR, B), lambda i: (i, 0)),
        out_shape=jax.ShapeDtypeStruct((N, B), x.dtype),
    )(xt, pe)
    return out.reshape(T, D, B).transpose(2, 0, 1)


# speedup vs baseline: 1.0956x; 1.0834x over previous
import jax
import jax.numpy as jnp
from jax.experimental import pallas as pl

_RR = 640


def _add_kernel(x_ref, pe_ref, o_ref):
    pev = pe_ref[0]
    g = _RR // 128
    rep = jnp.broadcast_to(pev[:, None, :], (g, 128, 128)).reshape(_RR, 128)
    sub = jax.lax.broadcasted_iota(jnp.int32, (_RR, 128), 0) % 128
    lane = jax.lax.broadcasted_iota(jnp.int32, (_RR, 128), 1)
    pe_col = jnp.sum(jnp.where(sub == lane, rep, 0.0), axis=1, keepdims=True)
    o_ref[0] = x_ref[0] + pe_col


def kernel(x, pos_emb):
    B, T, D = x.shape
    N = T * D
    G = N // _RR
    xt = x.transpose(1, 2, 0).reshape(G, _RR, B)
    pe = pos_emb.reshape(G, _RR // 128, 128)
    out = pl.pallas_call(
        _add_kernel,
        grid=(G,),
        in_specs=[
            pl.BlockSpec((1, _RR, B), lambda i: (i, 0, 0)),
            pl.BlockSpec((1, _RR // 128, 128), lambda i: (i, 0, 0)),
        ],
        out_specs=pl.BlockSpec((1, _RR, B), lambda i: (i, 0, 0)),
        out_shape=jax.ShapeDtypeStruct((G, _RR, B), x.dtype),
    )(xt, pe)
    return out.reshape(T, D, B).transpose(2, 0, 1)
